# baseline (speedup 1.0000x reference)
import functools

import jax
import jax.numpy as jnp
from jax import lax
from jax.experimental import pallas as pl
from jax.experimental.pallas import tpu as pltpu

ITEM_TILE = 2048
ROW_TILE = 256
K = 100
NEG = -3.0e38
SPAD = -1.0e30
def _matmul_body(u_ref, it_ref, out_ref, bmax_ref, *, n_items):
    t = pl.program_id(0)
    s = lax.dot_general(
        u_ref[...], it_ref[...],
        (((1,), (1,)), ((), ())),
        preferred_element_type=jnp.float32,
    )
    col = t * ITEM_TILE + lax.broadcasted_iota(jnp.int32, s.shape, 1)
    s = jnp.where(col < n_items, s, SPAD)
    out_ref[...] = s
    b = s
    for half in (1024, 512, 256, 128):
        b = jnp.maximum(b[:, :half], b[:, half:])
    bmax_ref[...] = b


def _scores_pallas(u, items_pad, n_items):
    q, d = u.shape
    n_pad = items_pad.shape[0]
    n_tiles = n_pad // ITEM_TILE
    grid = (n_tiles, q // ROW_TILE)
    return pl.pallas_call(
        functools.partial(_matmul_body, n_items=n_items),
        grid=grid,
        in_specs=[
            pl.BlockSpec((ROW_TILE, d), lambda t, r: (r, 0)),
            pl.BlockSpec((ITEM_TILE, d), lambda t, r: (t, 0)),
        ],
        out_specs=[
            pl.BlockSpec((ROW_TILE, ITEM_TILE), lambda t, r: (r, t)),
            pl.BlockSpec((ROW_TILE, 128), lambda t, r: (r, t)),
        ],
        out_shape=[
            jax.ShapeDtypeStruct((q, n_pad), jnp.float32),
            jax.ShapeDtypeStruct((q, n_tiles * 128), jnp.float32),
        ],
    )(u, items_pad)


def _peel(x, k, fold_to=None, with_vals=False, row_block=None):
    q, n = x.shape

    def body(x_ref, *out_refs):
        v = x_ref[...]
        m = n
        if fold_to is not None:
            while m > fold_to:
                m //= 2
                v = jnp.maximum(v[:, :m], v[:, m:])
        r = v.shape[0]
        col = lax.broadcasted_iota(jnp.int32, (r, m), 1)
        ocol = lax.broadcasted_iota(jnp.int32, (r, 128), 1)

        def step(i, carry):
            v, acc, accv = carry
            mx = jnp.max(v, axis=1, keepdims=True)
            am = jnp.min(jnp.where(v == mx, col, m), axis=1, keepdims=True)
            acc = jnp.where(ocol == i, am, acc)
            if with_vals:
                accv = jnp.where(ocol == i, mx, accv)
            v = jnp.where(col == am, NEG, v)
            return v, acc, accv

        _, acc, accv = lax.fori_loop(
            0, k, step,
            (v, jnp.zeros((r, 128), jnp.int32),
             jnp.full((r, 128), NEG, jnp.float32)), unroll=25)
        out_refs[0][...] = acc
        if with_vals:
            out_refs[1][...] = accv

    rb = row_block or q
    out_shape = [jax.ShapeDtypeStruct((q, 128), jnp.int32)]
    out_specs = [pl.BlockSpec((rb, 128), lambda r: (r, 0))]
    if with_vals:
        out_shape.append(jax.ShapeDtypeStruct((q, 128), jnp.float32))
        out_specs.append(pl.BlockSpec((rb, 128), lambda r: (r, 0)))
    res = pl.pallas_call(
        body,
        grid=(q // rb,),
        in_specs=[pl.BlockSpec((rb, n), lambda r: (r, 0))],
        out_specs=out_specs,
        out_shape=out_shape,
    )(x)
    return res if with_vals else (res[0],)


def _sc_user_gather(user_embs, uids):
    from jax.experimental.pallas import tpu_sc as plsc

    v, d = user_embs.shape
    b = uids.shape[0]
    info = plsc.get_sparse_core_info()
    nw = info.num_cores * info.num_subcores
    b_per_w = b // nw
    mesh = plsc.VectorSubcoreMesh(core_axis_name="c", subcore_axis_name="s")

    @functools.partial(
        pl.kernel, mesh=mesh,
        out_type=jax.ShapeDtypeStruct((b, d), jnp.float32),
        scratch_types=[
            pltpu.VMEM((b_per_w,), jnp.int32),
            pltpu.VMEM((b_per_w, d), jnp.float32),
            pltpu.SemaphoreType.DMA,
        ],
    )
    def k(table_hbm, idx_hbm, out_hbm, idx_v, rows_v, sem):
        wid = lax.axis_index("s") * info.num_cores + lax.axis_index("c")
        base = wid * b_per_w
        pltpu.sync_copy(idx_hbm.at[pl.ds(base, b_per_w)], idx_v)
        pltpu.async_copy(table_hbm.at[idx_v], rows_v, sem).wait()
        pltpu.sync_copy(rows_v, out_hbm.at[pl.ds(base, b_per_w)])

    return k(user_embs, uids.astype(jnp.int32))


def _pad_lanes(x, n_to):
    return jnp.pad(x, ((0, 0), (0, n_to - x.shape[1])),
                   constant_values=float(NEG))


def kernel(uids, topk, user_embs, item_embs):
    n_items, d = item_embs.shape
    q = uids.shape[0]
    n_pad = ((n_items + ITEM_TILE - 1) // ITEM_TILE) * ITEM_TILE
    items_pad = jnp.pad(item_embs, ((0, n_pad - n_items), (0, 0)))
    u = _sc_user_gather(user_embs, uids)
    scores, bmax = _scores_pallas(u, items_pad, n_items)
    bmax = _pad_lanes(bmax, 8192)

    (g_ids,) = _peel(bmax, K, fold_to=1024, row_block=512)
    g_ids = g_ids[:, :K]

    j16 = jnp.arange(8, dtype=jnp.int32) * 1024
    lids = (g_ids[:, :, None] + j16[None, None, :]).reshape(q, K * 8)
    cand1 = jnp.take_along_axis(bmax, lids, axis=1)
    (p1,) = _peel(_pad_lanes(cand1, 896), K)
    leaf = jnp.take_along_axis(lids, p1[:, :K], axis=1)

    base = (leaf // 128) * ITEM_TILE + (leaf % 128)
    j128 = jnp.arange(16, dtype=jnp.int32) * 128
    iidx = (base[:, :, None] + j128[None, None, :]).reshape(q, K * 16)
    cand2 = jnp.take_along_axis(scores, iidx, axis=1)
    p2, vals = _peel(_pad_lanes(cand2, 1664), K, with_vals=True)
    top_vals = vals[:, :K]
    top_idx = jnp.take_along_axis(iidx, p2[:, :K], axis=1)

    _, top_idx, top_vals = lax.sort(
        (-top_vals, top_idx, top_vals), dimension=1, num_keys=2)
    top_idx = top_idx + jnp.asarray(topk - topk, dtype=top_idx.dtype)
    return top_vals, top_idx


# speedup vs baseline: 1.0645x; 1.0645x over previous
import functools

import jax
import jax.numpy as jnp
from jax import lax
from jax.experimental import pallas as pl
from jax.experimental.pallas import tpu as pltpu

ITEM_TILE = 2048
ROW_TILE = 512
K = 100
NEG = -3.0e38
SPAD = -1.0e30
def _matmul_body(u_ref, it_ref, out_ref, bmax_ref, *, n_items):
    t = pl.program_id(0)
    s = lax.dot_general(
        u_ref[...], it_ref[...],
        (((1,), (1,)), ((), ())),
        preferred_element_type=jnp.float32,
    )
    col = t * ITEM_TILE + lax.broadcasted_iota(jnp.int32, s.shape, 1)
    s = jnp.where(col < n_items, s, SPAD)
    out_ref[...] = s
    b = s
    for half in (1024, 512, 256, 128):
        b = jnp.maximum(b[:, :half], b[:, half:])
    bmax_ref[...] = b


def _scores_pallas(u, items_pad, n_items):
    q, d = u.shape
    n_pad = items_pad.shape[0]
    n_tiles = n_pad // ITEM_TILE
    grid = (n_tiles, q // ROW_TILE)
    return pl.pallas_call(
        functools.partial(_matmul_body, n_items=n_items),
        grid=grid,
        in_specs=[
            pl.BlockSpec((ROW_TILE, d), lambda t, r: (r, 0)),
            pl.BlockSpec((ITEM_TILE, d), lambda t, r: (t, 0)),
        ],
        out_specs=[
            pl.BlockSpec((ROW_TILE, ITEM_TILE), lambda t, r: (r, t)),
            pl.BlockSpec((ROW_TILE, 128), lambda t, r: (r, t)),
        ],
        out_shape=[
            jax.ShapeDtypeStruct((q, n_pad), jnp.float32),
            jax.ShapeDtypeStruct((q, n_tiles * 128), jnp.float32),
        ],
    )(u, items_pad)


def _peel(x, k, fold_to=None, with_vals=False, row_block=None):
    q, n = x.shape

    def body(x_ref, *out_refs):
        v = x_ref[...]
        m = n
        if fold_to is not None:
            while m > fold_to:
                m //= 2
                v = jnp.maximum(v[:, :m], v[:, m:])
        r = v.shape[0]
        col = lax.broadcasted_iota(jnp.int32, (r, m), 1)
        ocol = lax.broadcasted_iota(jnp.int32, (r, 128), 1)

        def step(i, carry):
            v, acc, accv = carry
            mx = jnp.max(v, axis=1, keepdims=True)
            am = jnp.min(jnp.where(v == mx, col, m), axis=1, keepdims=True)
            acc = jnp.where(ocol == i, am, acc)
            if with_vals:
                accv = jnp.where(ocol == i, mx, accv)
            v = jnp.where(col == am, NEG, v)
            return v, acc, accv

        _, acc, accv = lax.fori_loop(
            0, k, step,
            (v, jnp.zeros((r, 128), jnp.int32),
             jnp.full((r, 128), NEG, jnp.float32)), unroll=10)
        out_refs[0][...] = acc
        if with_vals:
            out_refs[1][...] = accv

    rb = row_block or q
    out_shape = [jax.ShapeDtypeStruct((q, 128), jnp.int32)]
    out_specs = [pl.BlockSpec((rb, 128), lambda r: (r, 0))]
    if with_vals:
        out_shape.append(jax.ShapeDtypeStruct((q, 128), jnp.float32))
        out_specs.append(pl.BlockSpec((rb, 128), lambda r: (r, 0)))
    res = pl.pallas_call(
        body,
        grid=(q // rb,),
        in_specs=[pl.BlockSpec((rb, n), lambda r: (r, 0))],
        out_specs=out_specs,
        out_shape=out_shape,
    )(x)
    return res if with_vals else (res[0],)


def _sc_user_gather(user_embs, uids):
    from jax.experimental.pallas import tpu_sc as plsc

    v, d = user_embs.shape
    b = uids.shape[0]
    info = plsc.get_sparse_core_info()
    nw = info.num_cores * info.num_subcores
    b_per_w = b // nw
    mesh = plsc.VectorSubcoreMesh(core_axis_name="c", subcore_axis_name="s")

    @functools.partial(
        pl.kernel, mesh=mesh,
        out_type=jax.ShapeDtypeStruct((b, d), jnp.float32),
        scratch_types=[
            pltpu.VMEM((b_per_w,), jnp.int32),
            pltpu.VMEM((b_per_w, d), jnp.float32),
            pltpu.SemaphoreType.DMA,
        ],
    )
    def k(table_hbm, idx_hbm, out_hbm, idx_v, rows_v, sem):
        wid = lax.axis_index("s") * info.num_cores + lax.axis_index("c")
        base = wid * b_per_w
        pltpu.sync_copy(idx_hbm.at[pl.ds(base, b_per_w)], idx_v)
        pltpu.async_copy(table_hbm.at[idx_v], rows_v, sem).wait()
        pltpu.sync_copy(rows_v, out_hbm.at[pl.ds(base, b_per_w)])

    return k(user_embs, uids.astype(jnp.int32))


def _pad_lanes(x, n_to):
    return jnp.pad(x, ((0, 0), (0, n_to - x.shape[1])),
                   constant_values=float(NEG))


def kernel(uids, topk, user_embs, item_embs):
    n_items, d = item_embs.shape
    q = uids.shape[0]
    n_pad = ((n_items + ITEM_TILE - 1) // ITEM_TILE) * ITEM_TILE
    items_pad = jnp.pad(item_embs, ((0, n_pad - n_items), (0, 0)))
    u = _sc_user_gather(user_embs, uids)
    scores, bmax = _scores_pallas(u, items_pad, n_items)
    bmax = _pad_lanes(bmax, 8192)

    (g_ids,) = _peel(bmax, K, fold_to=1024, row_block=512)
    g_ids = g_ids[:, :K]

    j16 = jnp.arange(8, dtype=jnp.int32) * 1024
    lids = (g_ids[:, :, None] + j16[None, None, :]).reshape(q, K * 8)
    cand1 = jnp.take_along_axis(bmax, lids, axis=1)
    (p1,) = _peel(_pad_lanes(cand1, 896), K)
    leaf = jnp.take_along_axis(lids, p1[:, :K], axis=1)

    base = (leaf // 128) * ITEM_TILE + (leaf % 128)
    j128 = jnp.arange(16, dtype=jnp.int32) * 128
    iidx = (base[:, :, None] + j128[None, None, :]).reshape(q, K * 16)
    cand2 = jnp.take_along_axis(scores, iidx, axis=1)
    p2, vals = _peel(_pad_lanes(cand2, 1664), K, with_vals=True)
    top_vals = vals[:, :K]
    top_idx = jnp.take_along_axis(iidx, p2[:, :K], axis=1)

    _, top_idx, top_vals = lax.sort(
        (-top_vals, top_idx, top_vals), dimension=1, num_keys=2)
    top_idx = top_idx + jnp.asarray(topk - topk, dtype=top_idx.dtype)
    return top_vals, top_idx


# speedup vs baseline: 1.1128x; 1.0453x over previous
import functools

import jax
import jax.numpy as jnp
from jax import lax
from jax.experimental import pallas as pl
from jax.experimental.pallas import tpu as pltpu

ITEM_TILE = 2048
ROW_TILE = 1024
K = 100
NEG = -3.0e38
SPAD = -1.0e30
def _matmul_body(u_ref, it_ref, out_ref, bmax_ref, *, n_items):
    t = pl.program_id(0)
    s = lax.dot_general(
        u_ref[...], it_ref[...],
        (((1,), (1,)), ((), ())),
        preferred_element_type=jnp.float32,
    )
    col = t * ITEM_TILE + lax.broadcasted_iota(jnp.int32, s.shape, 1)
    s = jnp.where(col < n_items, s, SPAD)
    out_ref[...] = s
    b = s
    for half in (1024, 512, 256, 128):
        b = jnp.maximum(b[:, :half], b[:, half:])
    bmax_ref[...] = b


def _scores_pallas(u, items_pad, n_items):
    q, d = u.shape
    n_pad = items_pad.shape[0]
    n_tiles = n_pad // ITEM_TILE
    grid = (n_tiles, q // ROW_TILE)
    return pl.pallas_call(
        functools.partial(_matmul_body, n_items=n_items),
        grid=grid,
        in_specs=[
            pl.BlockSpec((ROW_TILE, d), lambda t, r: (r, 0)),
            pl.BlockSpec((ITEM_TILE, d), lambda t, r: (t, 0)),
        ],
        out_specs=[
            pl.BlockSpec((ROW_TILE, ITEM_TILE), lambda t, r: (r, t)),
            pl.BlockSpec((ROW_TILE, 128), lambda t, r: (r, t)),
        ],
        out_shape=[
            jax.ShapeDtypeStruct((q, n_pad), jnp.float32),
            jax.ShapeDtypeStruct((q, n_tiles * 128), jnp.float32),
        ],
    )(u, items_pad)


def _peel(x, k, fold_to=None, with_vals=False, row_block=None):
    q, n = x.shape

    def body(x_ref, *out_refs):
        v = x_ref[...]
        m = n
        if fold_to is not None:
            while m > fold_to:
                m //= 2
                v = jnp.maximum(v[:, :m], v[:, m:])
        r = v.shape[0]
        col = lax.broadcasted_iota(jnp.int32, (r, m), 1)
        ocol = lax.broadcasted_iota(jnp.int32, (r, 128), 1)

        def step(i, carry):
            v, acc, accv = carry
            mx = jnp.max(v, axis=1, keepdims=True)
            am = jnp.min(jnp.where(v == mx, col, m), axis=1, keepdims=True)
            acc = jnp.where(ocol == i, am, acc)
            if with_vals:
                accv = jnp.where(ocol == i, mx, accv)
            v = jnp.where(col == am, NEG, v)
            return v, acc, accv

        _, acc, accv = lax.fori_loop(
            0, k, step,
            (v, jnp.zeros((r, 128), jnp.int32),
             jnp.full((r, 128), NEG, jnp.float32)), unroll=10)
        out_refs[0][...] = acc
        if with_vals:
            out_refs[1][...] = accv

    rb = row_block or q
    out_shape = [jax.ShapeDtypeStruct((q, 128), jnp.int32)]
    out_specs = [pl.BlockSpec((rb, 128), lambda r: (r, 0))]
    if with_vals:
        out_shape.append(jax.ShapeDtypeStruct((q, 128), jnp.float32))
        out_specs.append(pl.BlockSpec((rb, 128), lambda r: (r, 0)))
    res = pl.pallas_call(
        body,
        grid=(q // rb,),
        in_specs=[pl.BlockSpec((rb, n), lambda r: (r, 0))],
        out_specs=out_specs,
        out_shape=out_shape,
    )(x)
    return res if with_vals else (res[0],)


def _sc_user_gather(user_embs, uids):
    from jax.experimental.pallas import tpu_sc as plsc

    v, d = user_embs.shape
    b = uids.shape[0]
    info = plsc.get_sparse_core_info()
    nw = info.num_cores * info.num_subcores
    b_per_w = b // nw
    mesh = plsc.VectorSubcoreMesh(core_axis_name="c", subcore_axis_name="s")

    @functools.partial(
        pl.kernel, mesh=mesh,
        out_type=jax.ShapeDtypeStruct((b, d), jnp.float32),
        scratch_types=[
            pltpu.VMEM((b_per_w,), jnp.int32),
            pltpu.VMEM((b_per_w, d), jnp.float32),
            pltpu.SemaphoreType.DMA,
        ],
    )
    def k(table_hbm, idx_hbm, out_hbm, idx_v, rows_v, sem):
        wid = lax.axis_index("s") * info.num_cores + lax.axis_index("c")
        base = wid * b_per_w
        pltpu.sync_copy(idx_hbm.at[pl.ds(base, b_per_w)], idx_v)
        pltpu.async_copy(table_hbm.at[idx_v], rows_v, sem).wait()
        pltpu.sync_copy(rows_v, out_hbm.at[pl.ds(base, b_per_w)])

    return k(user_embs, uids.astype(jnp.int32))


def _pad_lanes(x, n_to):
    return jnp.pad(x, ((0, 0), (0, n_to - x.shape[1])),
                   constant_values=float(NEG))


def kernel(uids, topk, user_embs, item_embs):
    n_items, d = item_embs.shape
    q = uids.shape[0]
    n_pad = ((n_items + ITEM_TILE - 1) // ITEM_TILE) * ITEM_TILE
    items_pad = jnp.pad(item_embs, ((0, n_pad - n_items), (0, 0)))
    u = _sc_user_gather(user_embs, uids)
    scores, bmax = _scores_pallas(u, items_pad, n_items)
    bmax = _pad_lanes(bmax, 8192)

    (g_ids,) = _peel(bmax, K, fold_to=1024, row_block=512)
    g_ids = g_ids[:, :K]

    j16 = jnp.arange(8, dtype=jnp.int32) * 1024
    lids = (g_ids[:, :, None] + j16[None, None, :]).reshape(q, K * 8)
    cand1 = jnp.take_along_axis(bmax, lids, axis=1)
    (p1,) = _peel(_pad_lanes(cand1, 896), K)
    leaf = jnp.take_along_axis(lids, p1[:, :K], axis=1)

    base = (leaf // 128) * ITEM_TILE + (leaf % 128)
    j128 = jnp.arange(16, dtype=jnp.int32) * 128
    iidx = (base[:, :, None] + j128[None, None, :]).reshape(q, K * 16)
    cand2 = jnp.take_along_axis(scores, iidx, axis=1)
    p2, vals = _peel(_pad_lanes(cand2, 1664), K, with_vals=True)
    top_vals = vals[:, :K]
    top_idx = jnp.take_along_axis(iidx, p2[:, :K], axis=1)

    _, top_idx, top_vals = lax.sort(
        (-top_vals, top_idx, top_vals), dimension=1, num_keys=2)
    top_idx = top_idx + jnp.asarray(topk - topk, dtype=top_idx.dtype)
    return top_vals, top_idx
